# baseline (speedup 1.0000x reference)
import jax
import jax.numpy as jnp
from jax.experimental import pallas as pl
from jax.experimental.pallas import tpu as pltpu

_B, _S, _A, _F, _HID, _OUT = 256, 50, 16, 16, 64, 64
_N = _B * _A
_G4 = 4 * _HID
_RH = _F
_RC = _F + _HID
_R1 = _F + 2 * _HID
_K = _R1 + 1


def _gclstm_kernel(x_ref, hide_ref, na_ref, W3_ref, wco_ref,
                   Wl_ref, Whi_ref, bl_ref, out_ref, XH_ref, C32_ref):
    XH_ref[...] = jnp.zeros((_K, _N), jnp.bfloat16)
    XH_ref[_R1:_K] = jnp.ones((1, _N), jnp.bfloat16)
    C32_ref[...] = jnp.zeros((_HID, _N), jnp.float32)
    W3 = W3_ref[...]
    wco = wco_ref[...]

    _NC = 4
    _CW = _N // _NC

    def step(t, carry):
        XH_ref[0:_F] = x_ref[t]
        for h in range(_NC):
            lo, hi = h * _CW, (h + 1) * _CW
            G = jnp.dot(W3, XH_ref[:, lo:hi], preferred_element_type=jnp.float32)
            gi = jnp.tanh(G[0:_HID]) * 0.5 + 0.5
            gf = jnp.tanh(G[_HID:2 * _HID]) * 0.5 + 0.5
            gt = jnp.tanh(G[2 * _HID:3 * _HID])
            C = C32_ref[:, lo:hi]
            Cn = gf * C + gi * gt
            go = jnp.tanh(G[3 * _HID:4 * _HID] + wco * Cn) * 0.5 + 0.5
            XH_ref[_RH:_RC, lo:hi] = (go * jnp.tanh(Cn)).astype(jnp.bfloat16)
            XH_ref[_RC:_R1, lo:hi] = Cn.astype(jnp.bfloat16)
            C32_ref[:, lo:hi] = Cn
        return carry

    jax.lax.fori_loop(0, _S, step, 0, unroll=10)

    na = na_ref[...]
    node = jax.lax.broadcasted_iota(jnp.int32, (_N, _B), 0)
    col_a = jax.lax.broadcasted_iota(jnp.int32, (_N, _B), 1) * _A
    inv = 1.0 / jnp.maximum(na.astype(jnp.float32), 1.0)
    sel = (node >= col_a) & (node < col_a + na)
    Mm = jnp.where(sel, inv, 0.0)
    res = jnp.dot(XH_ref[_RH:_RC].astype(jnp.float32), Mm,
                  preferred_element_type=jnp.float32)

    out_ref[...] = jnp.tanh(
        jnp.dot(Wl_ref[...], res, preferred_element_type=jnp.float32)
        + jnp.dot(Whi_ref[...], hide_ref[...], preferred_element_type=jnp.float32)
        + bl_ref[...])


def kernel(agent_obs, hideout_obs, timestep_obs, num_agents,
           W_i, Wh_i, bh_i, b_i, wc_i,
           W_f, Wh_f, bh_f, b_f, wc_f,
           W_c, Wh_c, bh_c, b_c,
           W_o, Wh_o, bh_o, b_o, wc_o,
           Wl_ag, bl_ag, Wr_ag, Wl_hi, bl_hi, Wr_hi):
    xT = jnp.transpose(agent_obs.astype(jnp.bfloat16), (1, 3, 0, 2)).reshape(_S, _F, _B * _A)
    W = jnp.concatenate([W_i, W_f, W_c, W_o], axis=1).T
    Wh = jnp.concatenate([Wh_i, Wh_f, Wh_c, Wh_o], axis=1).T
    z64 = jnp.zeros((_HID, _HID), jnp.float32)
    Wc = jnp.concatenate([jnp.diag(wc_i.reshape(-1)),
                          jnp.diag(wc_f.reshape(-1)),
                          z64, z64], axis=0)
    b = jnp.concatenate([b_i + bh_i, b_f + bh_f,
                         b_c + bh_c, b_o + bh_o]).reshape(_G4, 1)
    W3 = jnp.concatenate([W, Wh, Wc, b], axis=1)
    scale = jnp.concatenate([jnp.full((_HID,), 0.5, jnp.float32),
                             jnp.full((_HID,), 0.5, jnp.float32),
                             jnp.ones((_HID,), jnp.float32),
                             jnp.full((_HID,), 0.5, jnp.float32)]).reshape(_G4, 1)
    W3 = W3 * scale
    wco = wc_o.reshape(_HID, 1) * 0.5
    na2 = num_agents.reshape(1, _B).astype(jnp.int32)
    hideT = hideout_obs.T
    Wl = Wl_ag.T
    Whi = Wl_hi.T
    bl = (bl_ag + bl_hi).reshape(_OUT, 1)

    out_t = pl.pallas_call(
        _gclstm_kernel,
        out_shape=jax.ShapeDtypeStruct((_OUT, _B), jnp.float32),
        scratch_shapes=[pltpu.VMEM((_K, _N), jnp.bfloat16),
                        pltpu.VMEM((_HID, _N), jnp.float32)],
    )(xT, hideT, na2, W3.astype(jnp.bfloat16), wco, Wl, Whi, bl)
    return out_t.T


# speedup vs baseline: 1.2471x; 1.2471x over previous
import jax
import jax.numpy as jnp
from jax.experimental import pallas as pl
from jax.experimental.pallas import tpu as pltpu

_B, _S, _A, _F, _HID, _OUT = 256, 50, 16, 16, 64, 64
_N = _B * _A
_G4 = 4 * _HID
_AF = _A * _F
_SG = 10
_SB = _S // _SG
_RH = _F
_RC = _F + _HID
_R1 = _F + 2 * _HID
_K = _R1 + 1


def _gclstm_kernel(x_ref, hide_ref, na_ref, W3_ref, wco_ref,
                   Wl_ref, Whi_ref, bl_ref, out_ref, XH_ref):
    i = pl.program_id(0)

    @pl.when(i == 0)
    def _init():
        XH_ref[...] = jnp.zeros((_K, _N), jnp.float32)
        XH_ref[_R1:_K] = jnp.ones((1, _N), jnp.float32)

    W3 = W3_ref[...]
    wco = wco_ref[...]

    _NC = 4
    _CW = _N // _NC

    for j in range(_SB):
        t_x = jnp.transpose(x_ref[:, 0, j, :])
        for a in range(_A):
            XH_ref[0:_F, a * _B:(a + 1) * _B] = t_x[a * _F:(a + 1) * _F, :]
        for h in range(_NC):
            lo, hi = h * _CW, (h + 1) * _CW
            G = jnp.dot(W3, XH_ref[:, lo:hi], preferred_element_type=jnp.float32)
            gi = jnp.tanh(G[0:_HID]) * 0.5 + 0.5
            gf = jnp.tanh(G[_HID:2 * _HID]) * 0.5 + 0.5
            gt = jnp.tanh(G[2 * _HID:3 * _HID])
            C = XH_ref[_RC:_R1, lo:hi]
            Cn = gf * C + gi * gt
            go = jnp.tanh(G[3 * _HID:4 * _HID] + wco * Cn) * 0.5 + 0.5
            XH_ref[_RH:_RC, lo:hi] = go * jnp.tanh(Cn)
            XH_ref[_RC:_R1, lo:hi] = Cn

    @pl.when(i == _SG - 1)
    def _finish():
        na = na_ref[...]
        node = jax.lax.broadcasted_iota(jnp.int32, (_N, _B), 0)
        col = jax.lax.broadcasted_iota(jnp.int32, (_N, _B), 1)
        b_idx = jnp.bitwise_and(node, _B - 1)
        a_idx = node // _B
        inv = 1.0 / jnp.maximum(na.astype(jnp.float32), 1.0)
        sel = (b_idx == col) & (a_idx < na)
        Mm = jnp.where(sel, inv, 0.0)
        res = jnp.dot(XH_ref[_RH:_RC], Mm,
                      preferred_element_type=jnp.float32)
        out_ref[...] = jnp.tanh(
            jnp.dot(Wl_ref[...], res, preferred_element_type=jnp.float32)
            + jnp.dot(Whi_ref[...], hide_ref[...],
                      preferred_element_type=jnp.float32)
            + bl_ref[...])


def kernel(agent_obs, hideout_obs, timestep_obs, num_agents,
           W_i, Wh_i, bh_i, b_i, wc_i,
           W_f, Wh_f, bh_f, b_f, wc_f,
           W_c, Wh_c, bh_c, b_c,
           W_o, Wh_o, bh_o, b_o, wc_o,
           Wl_ag, bl_ag, Wr_ag, Wl_hi, bl_hi, Wr_hi):
    x3 = agent_obs.reshape(_B, _SG, _SB, _AF)
    W = jnp.concatenate([W_i, W_f, W_c, W_o], axis=1).T
    Wh = jnp.concatenate([Wh_i, Wh_f, Wh_c, Wh_o], axis=1).T
    z64 = jnp.zeros((_HID, _HID), jnp.float32)
    Wc = jnp.concatenate([jnp.diag(wc_i.reshape(-1)),
                          jnp.diag(wc_f.reshape(-1)),
                          z64, z64], axis=0)
    b = jnp.concatenate([b_i + bh_i, b_f + bh_f,
                         b_c + bh_c, b_o + bh_o]).reshape(_G4, 1)
    W3 = jnp.concatenate([W, Wh, Wc, b], axis=1)
    scale = jnp.concatenate([jnp.full((_HID,), 0.5, jnp.float32),
                             jnp.full((_HID,), 0.5, jnp.float32),
                             jnp.ones((_HID,), jnp.float32),
                             jnp.full((_HID,), 0.5, jnp.float32)]).reshape(_G4, 1)
    W3 = W3 * scale
    wco = wc_o.reshape(_HID, 1) * 0.5
    na2 = num_agents.reshape(1, _B).astype(jnp.int32)
    hideT = hideout_obs.T
    Wl = Wl_ag.T
    Whi = Wl_hi.T
    bl = (bl_ag + bl_hi).reshape(_OUT, 1)

    out_t = pl.pallas_call(
        _gclstm_kernel,
        grid=(_SG,),
        in_specs=[
            pl.BlockSpec((_B, 1, _SB, _AF), lambda i: (0, i, 0, 0)),
            pl.BlockSpec((2, _B), lambda i: (0, 0)),
            pl.BlockSpec((1, _B), lambda i: (0, 0)),
            pl.BlockSpec((_G4, _K), lambda i: (0, 0)),
            pl.BlockSpec((_HID, 1), lambda i: (0, 0)),
            pl.BlockSpec((_OUT, _HID), lambda i: (0, 0)),
            pl.BlockSpec((_OUT, 2), lambda i: (0, 0)),
            pl.BlockSpec((_OUT, 1), lambda i: (0, 0)),
        ],
        out_specs=pl.BlockSpec((_OUT, _B), lambda i: (0, 0)),
        out_shape=jax.ShapeDtypeStruct((_OUT, _B), jnp.float32),
        scratch_shapes=[pltpu.VMEM((_K, _N), jnp.float32)],
        compiler_params=pltpu.CompilerParams(
            dimension_semantics=("arbitrary",)),
    )(x3, hideT, na2, W3, wco, Wl, Whi, bl)
    return out_t.T
